# baseline (speedup 1.0000x reference)
import functools

import jax
import jax.numpy as jnp
from jax.experimental import pallas as pl
from jax.experimental.pallas import tpu as pltpu

V = 64
H = 384
W = 384
HW = H * W
B = 147456


def _body(pts_ref, pix_ref, coef_ref, out_ref):
    d = pts_ref[0]
    px = pix_ref[0:1, :]
    py = pix_ref[1:2, :]
    cf = coef_ref[...]
    for j in range(3):
        a = cf[:, j:j + 1]
        b = cf[:, 3 + j:4 + j]
        c = cf[:, 6 + j:7 + j]
        t = cf[:, 9 + j:10 + j]
        out_ref[j] = d * (a * px + b * py + c) + t


@jax.jit
def kernel(pts3d_cam, pixels, focals, pp, poses):
    ptsT = jnp.transpose(pts3d_cam, (2, 0, 1))
    pixT = jnp.transpose(pixels, (1, 0))
    fx = focals[:, 0:1]
    fy = focals[:, 1:2]
    a = poses[:, :3, 0] / fx
    b = poses[:, :3, 1] / fy
    c = poses[:, :3, 2] - a * pp[:, 0:1] - b * pp[:, 1:2]
    t = poses[:, :3, 3]
    coef = jnp.concatenate([a, b, c, t], axis=1)

    out = pl.pallas_call(
        _body,
        grid=(HW // B, V // 8),
        in_specs=[
            pl.BlockSpec((1, 8, B), lambda pb, vb: (2, vb, pb)),
            pl.BlockSpec((2, B), lambda pb, vb: (0, pb)),
            pl.BlockSpec((8, 12), lambda pb, vb: (vb, 0)),
        ],
        out_specs=pl.BlockSpec((3, 8, B), lambda pb, vb: (0, vb, pb)),
        out_shape=jax.ShapeDtypeStruct((3, V, HW), jnp.float32),
        compiler_params=pltpu.CompilerParams(
            dimension_semantics=("parallel", "parallel")),
    )(ptsT, pixT, coef)
    return jnp.transpose(out, (1, 2, 0))


# speedup vs baseline: 1.0173x; 1.0173x over previous
import jax
import jax.numpy as jnp
from jax.experimental import pallas as pl
from jax.experimental.pallas import tpu as pltpu

V = 64
H = 384
W = 384
HW = H * W
B = 73728


def _body(pts_ref, pix_ref, coef_ref, out_ref):
    d = pts_ref[0]
    px = pix_ref[0:1, :]
    py = pix_ref[1:2, :]
    cf = coef_ref[...]
    for j in range(3):
        a = cf[:, j:j + 1]
        b = cf[:, 3 + j:4 + j]
        c = cf[:, 6 + j:7 + j]
        t = cf[:, 9 + j:10 + j]
        out_ref[j] = d * (a * px + b * py + c) + t


@jax.jit
def kernel(pts3d_cam, pixels, focals, pp, poses):
    ptsT = jnp.transpose(pts3d_cam, (2, 0, 1))
    pixT = jnp.transpose(pixels, (1, 0))
    fx = focals[:, 0:1]
    fy = focals[:, 1:2]
    a = poses[:, :3, 0] / fx
    b = poses[:, :3, 1] / fy
    c = poses[:, :3, 2] - a * pp[:, 0:1] - b * pp[:, 1:2]
    t = poses[:, :3, 3]
    coef = jnp.concatenate([a, b, c, t], axis=1)

    out = pl.pallas_call(
        _body,
        grid=(HW // B, V // 8),
        in_specs=[
            pl.BlockSpec((1, 8, B), lambda pb, vb: (2, vb, pb)),
            pl.BlockSpec((2, B), lambda pb, vb: (0, pb)),
            pl.BlockSpec((8, 12), lambda pb, vb: (vb, 0)),
        ],
        out_specs=pl.BlockSpec((3, 8, B), lambda pb, vb: (0, vb, pb)),
        out_shape=jax.ShapeDtypeStruct((3, V, HW), jnp.float32),
        compiler_params=pltpu.CompilerParams(
            dimension_semantics=("parallel", "parallel")),
    )(ptsT, pixT, coef)
    return jnp.transpose(out, (1, 2, 0))
